# baseline (speedup 1.0000x reference)
import functools

import jax
import jax.numpy as jnp
from jax import lax
from jax.experimental import pallas as pl
from jax.experimental.pallas import tpu as pltpu
from jax.experimental.pallas import tpu_sc as plsc

NC = 2
NS = 16
LANES = 16
CHUNK = 128
NBUF = 4
HD = 64


def _sc_mesh():
    return plsc.VectorSubcoreMesh(
        core_axis_name="c", subcore_axis_name="s", num_cores=NC, num_subcores=NS
    )


def _splat(s, dtype=jnp.float32):
    return jnp.full((LANES,), s, dtype)


def _sc_body(x_hbm, ei_hbm, h2_hbm, g0_hbm, g1_hbm,
             rowv, colv, gbuf, degv, disv, dinvv, idr,
             acc_sh, deg_sh, sem0, sem1, sem2, sem3, *, n_real):
    sems = (sem0, sem1, sem2, sem3)
    cid = lax.axis_index("c")
    sid = lax.axis_index("s")
    cpt = rowv.shape[0]
    npad = acc_sh.shape[0]
    rpt = npad // NS
    nck = rpt // CHUNK
    base = sid * rpt

    ec = ei_hbm.shape[1]
    bt = ec // cpt
    rem = ec % cpt

    @pl.when(sid < bt)
    def _():
        pltpu.async_copy(ei_hbm.at[0, pl.ds(sid * cpt, cpt)], rowv, sem0)
        pltpu.async_copy(ei_hbm.at[1, pl.ds(sid * cpt, cpt)], colv, sem1)

    @pl.when(sid >= bt)
    def _():
        if rem > 0:
            pltpu.async_copy(
                ei_hbm.at[0, pl.ds(bt * cpt, rem)], rowv.at[pl.ds(0, rem)], sem0
            )
            pltpu.async_copy(
                ei_hbm.at[1, pl.ds(bt * cpt, rem)], colv.at[pl.ds(0, rem)], sem1
            )

        def _fill(rr, _):
            for j in range(CHUNK // LANES):
                vals = _splat(n_real, jnp.int32) + (
                    (jnp.arange(LANES, dtype=jnp.int32)
                     + _splat(rr * CHUNK + j * LANES, jnp.int32)) & 127
                )
                rowv[rr, pl.ds(j * LANES, LANES)] = vals
                colv[rr, pl.ds(j * LANES, LANES)] = vals
            return 0

        lax.fori_loop(rem, cpt, _fill, 0)

    zero16 = jnp.zeros((LANES,), jnp.float32)

    def _idr(k, _):
        def _idrj(j, _):
            idr[k, pl.ds(j * LANES, LANES)] = (
                jnp.arange(LANES, dtype=jnp.int32)
                + _splat(k * CHUNK + j * LANES, jnp.int32)
            )
            return 0

        lax.fori_loop(0, CHUNK // LANES, _idrj, 0)
        return 0

    lax.fori_loop(0, idr.shape[0], _idr, 0)

    def _zd(i, _):
        degv[i] = zero16
        return 0

    lax.fori_loop(0, degv.shape[0], _zd, 0)

    @pl.when(sid < bt)
    def _():
        pltpu.make_async_copy(
            ei_hbm.at[0, pl.ds(sid * cpt, cpt)], rowv, sem0
        ).wait()
        pltpu.make_async_copy(
            ei_hbm.at[1, pl.ds(sid * cpt, cpt)], colv, sem1
        ).wait()

    if rem > 0:
        @pl.when(sid >= bt)
        def _():
            pltpu.make_async_copy(
                ei_hbm.at[0, pl.ds(bt * cpt, rem)], rowv.at[pl.ds(0, rem)], sem0
            ).wait()
            pltpu.make_async_copy(
                ei_hbm.at[1, pl.ds(bt * cpt, rem)], colv.at[pl.ds(0, rem)], sem1
            ).wait()

    @pl.when(sid == 0)
    def _():
        pltpu.sync_copy(degv, deg_sh)

    ones = jnp.ones((LANES,), jnp.float32)

    def _hist(c):
        for j in range(CHUNK // LANES):
            cvec = colv[c, pl.ds(j * LANES, LANES)]
            plsc.addupdate_scatter(
                degv, [lax.shift_right_logical(cvec, 4), cvec & 15], ones
            )

    with jax.named_scope("p0_hist"):
        plsc.parallel_loop(0, cpt, unroll=2)(_hist)
    plsc.subcore_barrier()

    nred = degv.shape[0] // CHUNK
    for k in range(nred):
        pltpu.sync_copy(
            degv.at[pl.ds(k * CHUNK, CHUNK)], deg_sh.at[idr.at[k]], add=True
        )
    plsc.subcore_barrier()

    nrow16 = rpt // LANES
    pltpu.sync_copy(deg_sh.at[pl.ds(sid * nrow16, nrow16)], degv.at[pl.ds(0, nrow16)])

    def _newton(t, _):
        d = degv[t] + 1.0
        i = plsc.bitcast(d, jnp.int32)
        i = _splat(0x5F3759DF, jnp.int32) - lax.shift_right_logical(i, 1)
        y = plsc.bitcast(i, jnp.float32)
        for _ in range(3):
            y = y * (1.5 - 0.5 * d * y * y)
        disv[pl.ds(t * LANES, LANES)] = y
        dinvv[pl.ds(t * LANES, LANES)] = y * y
        return 0

    lax.fori_loop(0, nrow16, _newton, 0)

    half = pl.ds(cid * HD, HD)

    def _scale_phase(src_hbm, dst_hbm, svec, src_is_acc, wide_dst, acc_store):
        def _load(kk):
            sl = pl.ds(base + kk * CHUNK, CHUNK)
            if src_is_acc:
                return pltpu.make_async_copy(acc_sh.at[sl], gbuf.at[0], sem0)
            return pltpu.make_async_copy(src_hbm.at[sl, half], gbuf.at[0], sem0)

        def _store(kk):
            sl = pl.ds(base + kk * CHUNK, CHUNK)
            if wide_dst:
                return pltpu.make_async_copy(gbuf.at[2], dst_hbm.at[sl, half], sem2)
            return pltpu.make_async_copy(gbuf.at[2], dst_hbm.at[cid, sl], sem2)

        def _astore(kk):
            sl = pl.ds(base + kk * CHUNK, CHUNK)
            return pltpu.make_async_copy(gbuf.at[2], acc_sh.at[sl], sem3)

        _load(0).start()

        def _iter(kk, _):
            _load(kk).wait()

            @pl.when(kk > 0)
            def _():
                _store(kk - 1).wait()
                if acc_store:
                    _astore(kk - 1).wait()

            def _row16(t):
                dvec = svec[pl.ds(kk * CHUNK + t * LANES, LANES)]
                for l in range(LANES):
                    s = _splat(dvec[l])
                    r = t * LANES + l
                    for j in range(HD // LANES):
                        cs = pl.ds(j * LANES, LANES)
                        gbuf[2, r, cs] = gbuf[0, r, cs] * s

            plsc.parallel_loop(0, CHUNK // LANES, unroll=2)(_row16)
            _store(kk).start()
            if acc_store:
                _astore(kk).start()

            @pl.when(kk + 1 < nck)
            def _():
                _load(kk + 1).start()

            return 0

        lax.fori_loop(0, nck, _iter, 0)
        _store(nck - 1).wait()
        if acc_store:
            _astore(nck - 1).wait()

    with jax.named_scope("p1_scale"):
        _scale_phase(x_hbm, g0_hbm, disv, False, False, True)
    plsc.subcore_barrier()

    def _hop(src_hbm):
        def gather(cc, bb):
            pltpu.async_copy(src_hbm.at[cid].at[rowv.at[cc]], gbuf.at[bb], sems[bb])

        def gwait(cc, bb):
            pltpu.make_async_copy(
                src_hbm.at[cid].at[rowv.at[cc]], gbuf.at[bb], sems[bb]
            ).wait()

        gather(0, 0)
        gather(1, 1)

        def body(k, _):
            for b in range(NBUF):
                c = k * NBUF + b
                gwait(c, b)

                @pl.when(c + 2 < cpt)
                def _():
                    gather(c + 2, (b + 2) % NBUF)

                pltpu.sync_copy(gbuf.at[b], acc_sh.at[colv.at[c]], add=True)
            return 0

        lax.fori_loop(0, cpt // NBUF, body, 0)
        plsc.subcore_barrier()

    with jax.named_scope("p2_hop1"):
        _hop(g0_hbm)

    with jax.named_scope("p3_combine"):
        _scale_phase(None, g1_hbm, dinvv, True, False, True)
    plsc.subcore_barrier()

    with jax.named_scope("p4_hop2"):
        _hop(g1_hbm)

    with jax.named_scope("p5_out"):
        _scale_phase(None, h2_hbm, disv, True, True, False)


def _make_sc_kernel(npad, cpt, n_real):
    shp = jax.ShapeDtypeStruct((NC, npad, HD), jnp.float32)
    return pl.kernel(
        functools.partial(_sc_body, n_real=n_real),
        out_type=(
            jax.ShapeDtypeStruct((npad, 2 * HD), jnp.float32),
            shp,
            shp,
        ),
        mesh=_sc_mesh(),
        compiler_params=pltpu.CompilerParams(
            needs_layout_passes=False, use_tc_tiling_on_sc=False
        ),
        scratch_types=[
            pltpu.VMEM((cpt, CHUNK), jnp.int32),
            pltpu.VMEM((cpt, CHUNK), jnp.int32),
            pltpu.VMEM((NBUF, CHUNK, HD), jnp.float32),
            pltpu.VMEM((npad // LANES, LANES), jnp.float32),
            pltpu.VMEM((npad // NS,), jnp.float32),
            pltpu.VMEM((npad // NS,), jnp.float32),
            pltpu.VMEM((npad // LANES // CHUNK, CHUNK), jnp.int32),
            pltpu.VMEM_SHARED((npad, HD), jnp.float32),
            pltpu.VMEM_SHARED((npad // LANES, LANES), jnp.float32),
            pltpu.SemaphoreType.DMA,
            pltpu.SemaphoreType.DMA,
            pltpu.SemaphoreType.DMA,
            pltpu.SemaphoreType.DMA,
        ],
    )


def _mm_body(h2_ref, wt_ref, b_ref, out_ref):
    out_ref[...] = (
        jnp.dot(h2_ref[...], wt_ref[...], preferred_element_type=jnp.float32)
        + b_ref[...]
    )


def _mm_call(h2, wt, bp, n, n_cls, d_feat):
    npad = h2.shape[0]
    blk = 2048
    out = pl.pallas_call(
        _mm_body,
        grid=(npad // blk,),
        in_specs=[
            pl.BlockSpec((blk, d_feat), lambda i: (i, 0)),
            pl.BlockSpec((d_feat, n_cls), lambda i: (0, 0)),
            pl.BlockSpec((1, n_cls), lambda i: (0, 0)),
        ],
        out_specs=pl.BlockSpec((blk, n_cls), lambda i: (i, 0)),
        out_shape=jax.ShapeDtypeStruct((npad, n_cls), jnp.float32),
        compiler_params=pltpu.CompilerParams(
            dimension_semantics=("arbitrary",)
        ),
    )(h2, wt, bp)
    return out[:n]


def kernel(x, edge_index, W, b):
    n, d_feat = x.shape
    n_cls = W.shape[0]
    e = edge_index.shape[1]

    npad = ((n + 128 + 512 - 1) // 512) * 512
    unit = NS * CHUNK * NBUF
    epad = ((e + unit - 1) // unit) * unit
    cpt = epad // (NS * CHUNK)

    if e % CHUNK:
        pad1 = CHUNK - e % CHUNK
        edge_index = jnp.concatenate(
            [edge_index, jnp.full((2, pad1), n, edge_index.dtype)], axis=1
        )
        e += pad1
    ei = edge_index.astype(jnp.int32).reshape(2, e // CHUNK, CHUNK)

    xs = jnp.zeros((npad, d_feat), jnp.float32).at[:n].set(x)
    wt = W.T
    bp = b.reshape(1, n_cls)

    h2, _, _ = _make_sc_kernel(npad, cpt, n)(xs, ei)
    return _mm_call(h2, wt, bp, n, n_cls, d_feat)


# speedup vs baseline: 1.0018x; 1.0018x over previous
import functools

import jax
import jax.numpy as jnp
from jax import lax
from jax.experimental import pallas as pl
from jax.experimental.pallas import tpu as pltpu
from jax.experimental.pallas import tpu_sc as plsc

NC = 2
NS = 16
LANES = 16
CHUNK = 128
NBUF = 4
HD = 64


def _sc_mesh():
    return plsc.VectorSubcoreMesh(
        core_axis_name="c", subcore_axis_name="s", num_cores=NC, num_subcores=NS
    )


def _splat(s, dtype=jnp.float32):
    return jnp.full((LANES,), s, dtype)


def _sc_body(x_hbm, ei_hbm, h2_hbm, g0_hbm, g1_hbm,
             rowv, colv, gbuf, degv, disv, dinvv, idr,
             acc_sh, deg_sh, sem0, sem1, sem2, sem3, *, n_real):
    sems = (sem0, sem1, sem2, sem3)
    cid = lax.axis_index("c")
    sid = lax.axis_index("s")
    cpt = rowv.shape[0]
    npad = acc_sh.shape[0]
    rpt = npad // NS
    nck = rpt // CHUNK
    base = sid * rpt

    ec = ei_hbm.shape[1]
    bt = ec // cpt
    rem = ec % cpt

    @pl.when(sid < bt)
    def _():
        pltpu.async_copy(ei_hbm.at[0, pl.ds(sid * cpt, cpt)], rowv, sem0)
        pltpu.async_copy(ei_hbm.at[1, pl.ds(sid * cpt, cpt)], colv, sem1)

    if rem > 0:
        @pl.when(sid == bt)
        def _():
            pltpu.async_copy(
                ei_hbm.at[0, pl.ds(bt * cpt, rem)], rowv.at[pl.ds(0, rem)], sem0
            )
            pltpu.async_copy(
                ei_hbm.at[1, pl.ds(bt * cpt, rem)], colv.at[pl.ds(0, rem)], sem1
            )

    @pl.when(sid >= bt)
    def _():
        def _fill(rr, _):
            for j in range(CHUNK // LANES):
                vals = _splat(n_real, jnp.int32) + (
                    (jnp.arange(LANES, dtype=jnp.int32)
                     + _splat(rr * CHUNK + j * LANES, jnp.int32)) & 127
                )
                rowv[rr, pl.ds(j * LANES, LANES)] = vals
                colv[rr, pl.ds(j * LANES, LANES)] = vals
            return 0

        lax.fori_loop(jnp.where(sid == bt, rem, 0), cpt, _fill, 0)

    zero16 = jnp.zeros((LANES,), jnp.float32)

    def _idr(k, _):
        def _idrj(j, _):
            idr[k, pl.ds(j * LANES, LANES)] = (
                jnp.arange(LANES, dtype=jnp.int32)
                + _splat(k * CHUNK + j * LANES, jnp.int32)
            )
            return 0

        lax.fori_loop(0, CHUNK // LANES, _idrj, 0)
        return 0

    lax.fori_loop(0, idr.shape[0], _idr, 0)

    def _zd(i, _):
        degv[i] = zero16
        return 0

    lax.fori_loop(0, degv.shape[0], _zd, 0)

    @pl.when(sid < bt)
    def _():
        pltpu.make_async_copy(
            ei_hbm.at[0, pl.ds(sid * cpt, cpt)], rowv, sem0
        ).wait()
        pltpu.make_async_copy(
            ei_hbm.at[1, pl.ds(sid * cpt, cpt)], colv, sem1
        ).wait()

    if rem > 0:
        @pl.when(sid == bt)
        def _():
            pltpu.make_async_copy(
                ei_hbm.at[0, pl.ds(bt * cpt, rem)], rowv.at[pl.ds(0, rem)], sem0
            ).wait()
            pltpu.make_async_copy(
                ei_hbm.at[1, pl.ds(bt * cpt, rem)], colv.at[pl.ds(0, rem)], sem1
            ).wait()

    @pl.when(sid == 0)
    def _():
        pltpu.sync_copy(degv, deg_sh)

    ones = jnp.ones((LANES,), jnp.float32)

    def _hist(c):
        for j in range(CHUNK // LANES):
            cvec = colv[c, pl.ds(j * LANES, LANES)]
            plsc.addupdate_scatter(
                degv, [lax.shift_right_logical(cvec, 4), cvec & 15], ones
            )

    with jax.named_scope("p0_hist"):
        plsc.parallel_loop(0, cpt, unroll=2)(_hist)
    plsc.subcore_barrier()

    nred = degv.shape[0] // CHUNK
    for k in range(nred):
        pltpu.sync_copy(
            degv.at[pl.ds(k * CHUNK, CHUNK)], deg_sh.at[idr.at[k]], add=True
        )
    plsc.subcore_barrier()

    nrow16 = rpt // LANES
    pltpu.sync_copy(deg_sh.at[pl.ds(sid * nrow16, nrow16)], degv.at[pl.ds(0, nrow16)])

    def _newton(t, _):
        d = degv[t] + 1.0
        i = plsc.bitcast(d, jnp.int32)
        i = _splat(0x5F3759DF, jnp.int32) - lax.shift_right_logical(i, 1)
        y = plsc.bitcast(i, jnp.float32)
        for _ in range(3):
            y = y * (1.5 - 0.5 * d * y * y)
        disv[pl.ds(t * LANES, LANES)] = y
        dinvv[pl.ds(t * LANES, LANES)] = y * y
        return 0

    lax.fori_loop(0, nrow16, _newton, 0)

    half = pl.ds(cid * HD, HD)

    def _scale_phase(src_hbm, dst_hbm, svec, src_is_acc, wide_dst, acc_store):
        def _load(kk):
            sl = pl.ds(base + kk * CHUNK, CHUNK)
            if src_is_acc:
                return pltpu.make_async_copy(acc_sh.at[sl], gbuf.at[0], sem0)
            return pltpu.make_async_copy(src_hbm.at[sl, half], gbuf.at[0], sem0)

        def _store(kk):
            sl = pl.ds(base + kk * CHUNK, CHUNK)
            if wide_dst:
                return pltpu.make_async_copy(gbuf.at[2], dst_hbm.at[sl, half], sem2)
            return pltpu.make_async_copy(gbuf.at[2], dst_hbm.at[cid, sl], sem2)

        def _astore(kk):
            sl = pl.ds(base + kk * CHUNK, CHUNK)
            return pltpu.make_async_copy(gbuf.at[2], acc_sh.at[sl], sem3)

        _load(0).start()

        def _iter(kk, _):
            _load(kk).wait()

            @pl.when(kk > 0)
            def _():
                _store(kk - 1).wait()
                if acc_store:
                    _astore(kk - 1).wait()

            def _row16(t):
                dvec = svec[pl.ds(kk * CHUNK + t * LANES, LANES)]
                for l in range(LANES):
                    s = _splat(dvec[l])
                    r = t * LANES + l
                    for j in range(HD // LANES):
                        cs = pl.ds(j * LANES, LANES)
                        gbuf[2, r, cs] = gbuf[0, r, cs] * s

            plsc.parallel_loop(0, CHUNK // LANES, unroll=2)(_row16)
            _store(kk).start()
            if acc_store:
                _astore(kk).start()

            @pl.when(kk + 1 < nck)
            def _():
                _load(kk + 1).start()

            return 0

        lax.fori_loop(0, nck, _iter, 0)
        _store(nck - 1).wait()
        if acc_store:
            _astore(nck - 1).wait()

    with jax.named_scope("p1_scale"):
        _scale_phase(x_hbm, g0_hbm, disv, False, False, True)
    plsc.subcore_barrier()

    def _hop(src_hbm):
        def gather(cc, bb):
            pltpu.async_copy(src_hbm.at[cid].at[rowv.at[cc]], gbuf.at[bb], sems[bb])

        def gwait(cc, bb):
            pltpu.make_async_copy(
                src_hbm.at[cid].at[rowv.at[cc]], gbuf.at[bb], sems[bb]
            ).wait()

        gather(0, 0)
        gather(1, 1)

        def body(k, _):
            for b in range(NBUF):
                c = k * NBUF + b
                gwait(c, b)

                @pl.when(c + 2 < cpt)
                def _():
                    gather(c + 2, (b + 2) % NBUF)

                pltpu.sync_copy(gbuf.at[b], acc_sh.at[colv.at[c]], add=True)
            return 0

        lax.fori_loop(0, cpt // NBUF, body, 0)
        plsc.subcore_barrier()

    with jax.named_scope("p2_hop1"):
        _hop(g0_hbm)

    with jax.named_scope("p3_combine"):
        _scale_phase(None, g1_hbm, dinvv, True, False, True)
    plsc.subcore_barrier()

    with jax.named_scope("p4_hop2"):
        _hop(g1_hbm)

    with jax.named_scope("p5_out"):
        _scale_phase(None, h2_hbm, disv, True, True, False)


def _make_sc_kernel(npad, cpt, n_real):
    shp = jax.ShapeDtypeStruct((NC, npad, HD), jnp.float32)
    return pl.kernel(
        functools.partial(_sc_body, n_real=n_real),
        out_type=(
            jax.ShapeDtypeStruct((npad, 2 * HD), jnp.float32),
            shp,
            shp,
        ),
        mesh=_sc_mesh(),
        compiler_params=pltpu.CompilerParams(
            needs_layout_passes=False, use_tc_tiling_on_sc=False
        ),
        scratch_types=[
            pltpu.VMEM((cpt, CHUNK), jnp.int32),
            pltpu.VMEM((cpt, CHUNK), jnp.int32),
            pltpu.VMEM((NBUF, CHUNK, HD), jnp.float32),
            pltpu.VMEM((npad // LANES, LANES), jnp.float32),
            pltpu.VMEM((npad // NS,), jnp.float32),
            pltpu.VMEM((npad // NS,), jnp.float32),
            pltpu.VMEM((npad // LANES // CHUNK, CHUNK), jnp.int32),
            pltpu.VMEM_SHARED((npad, HD), jnp.float32),
            pltpu.VMEM_SHARED((npad // LANES, LANES), jnp.float32),
            pltpu.SemaphoreType.DMA,
            pltpu.SemaphoreType.DMA,
            pltpu.SemaphoreType.DMA,
            pltpu.SemaphoreType.DMA,
        ],
    )


def _mm_body(h2_ref, wt_ref, b_ref, out_ref):
    out_ref[...] = (
        jnp.dot(h2_ref[...], wt_ref[...], preferred_element_type=jnp.float32)
        + b_ref[...]
    )


def _mm_call(h2, wt, bp, n, n_cls, d_feat):
    blk = 2048
    grid = (n + blk - 1) // blk
    return pl.pallas_call(
        _mm_body,
        grid=(grid,),
        in_specs=[
            pl.BlockSpec((blk, d_feat), lambda i: (i, 0)),
            pl.BlockSpec((d_feat, n_cls), lambda i: (0, 0)),
            pl.BlockSpec((1, n_cls), lambda i: (0, 0)),
        ],
        out_specs=pl.BlockSpec((blk, n_cls), lambda i: (i, 0)),
        out_shape=jax.ShapeDtypeStruct((n, n_cls), jnp.float32),
        compiler_params=pltpu.CompilerParams(
            dimension_semantics=("arbitrary",)
        ),
    )(h2, wt, bp)


def kernel(x, edge_index, W, b):
    n, d_feat = x.shape
    n_cls = W.shape[0]
    e = edge_index.shape[1]

    npad = ((n + 128 + 512 - 1) // 512) * 512
    unit = NS * CHUNK * NBUF
    epad = ((e + unit - 1) // unit) * unit
    cpt = epad // (NS * CHUNK)

    if e % CHUNK:
        pad1 = CHUNK - e % CHUNK
        edge_index = jnp.concatenate(
            [edge_index, jnp.full((2, pad1), n, edge_index.dtype)], axis=1
        )
        e += pad1
    ei = edge_index.astype(jnp.int32).reshape(2, e // CHUNK, CHUNK)

    xs = jnp.zeros((npad, d_feat), jnp.float32).at[:n].set(x)
    wt = W.T
    bp = b.reshape(1, n_cls)

    h2, _, _ = _make_sc_kernel(npad, cpt, n)(xs, ei)
    return _mm_call(h2, wt, bp, n, n_cls, d_feat)
